# baseline (speedup 1.0000x reference)
import dataclasses
import jax
import jax.numpy as jnp
from jax import lax
from jax.experimental import pallas as pl
from jax.experimental.pallas import tpu as pltpu
from jax.experimental.pallas import tpu_sc as plsc

_NPIX = float(512 * 512)
_NW = 32
_SPAN = (512 * 512) // _NW


def _sc_hist(o_hbm, t_hbm, out_hbm, b0, b1, b2, b3, tb,
             bins, bins1, bins2, bins3, sem):
    allbins = [bins, bins1, bins2, bins3]
    c = lax.axis_index("c")
    s = lax.axis_index("s")
    w = s * 2 + c
    base = w * _SPAN
    zeros16 = jnp.zeros((16,), jnp.int32)
    ones16 = jnp.ones((16,), jnp.int32)
    lanes = jax.lax.iota(jnp.int32, 16)

    @pl.loop(0, 8)
    def _batch(bi):
        @pl.loop(0, 16)
        def _z(j):
            bins.at[j][...] = zeros16
            bins1.at[j][...] = zeros16
            bins2.at[j][...] = zeros16
            bins3.at[j][...] = zeros16

        cp0 = pltpu.async_copy(o_hbm.at[bi, 0, pl.ds(base, _SPAN)], b0, sem)
        cp1 = pltpu.async_copy(o_hbm.at[bi, 1, pl.ds(base, _SPAN)], b1, sem)
        cp2 = pltpu.async_copy(o_hbm.at[bi, 2, pl.ds(base, _SPAN)], b2, sem)
        cp3 = pltpu.async_copy(o_hbm.at[bi, 3, pl.ds(base, _SPAN)], b3, sem)
        cp4 = pltpu.async_copy(t_hbm.at[bi, pl.ds(base, _SPAN)], tb, sem)
        cp0.wait()
        cp1.wait()
        cp2.wait()
        cp3.wait()
        cp4.wait()

        @pl.loop(0, _SPAN, step=64)
        def _vec(i):
            for u in range(4):
                sl = pl.ds(i + u * 16, 16)
                o0 = b0.at[sl][...]
                o1 = b1.at[sl][...]
                o2 = b2.at[sl][...]
                o3 = b3.at[sl][...]
                tv = tb.at[sl][...]
                gt1 = o1 > o0
                gt3 = o3 > o2
                gtb = jnp.maximum(o2, o3) > jnp.maximum(o0, o1)
                idx = jnp.where(gtb,
                                jnp.where(gt3, jnp.int32(3), jnp.int32(2)),
                                jnp.where(gt1, jnp.int32(1), jnp.int32(0)))
                v = (idx << 2) | tv
                plsc.addupdate_scatter(allbins[u], [lanes, v], ones16)

        @pl.loop(0, 16)
        def _fold(j):
            bins.at[j][...] = (bins.at[j][...] + bins1.at[j][...]
                               + bins2.at[j][...] + bins3.at[j][...])

        pltpu.async_copy(bins, out_hbm.at[w, bi], sem).wait()


def _make_sc_kernel():
    cp = pltpu.CompilerParams()
    if "needs_layout_passes" in pltpu.CompilerParams.__dataclass_fields__:
        cp = dataclasses.replace(cp, needs_layout_passes=False)
    mesh = plsc.VectorSubcoreMesh(core_axis_name="c", subcore_axis_name="s")
    return pl.kernel(
        _sc_hist,
        out_type=jax.ShapeDtypeStruct((_NW, 8, 16, 16), jnp.int32),
        mesh=mesh,
        scratch_types=[
            pltpu.VMEM((_SPAN,), jnp.float32),
            pltpu.VMEM((_SPAN,), jnp.float32),
            pltpu.VMEM((_SPAN,), jnp.float32),
            pltpu.VMEM((_SPAN,), jnp.float32),
            pltpu.VMEM((_SPAN,), jnp.int32),
            pltpu.VMEM((16, 16), jnp.int32),
            pltpu.VMEM((16, 16), jnp.int32),
            pltpu.VMEM((16, 16), jnp.int32),
            pltpu.VMEM((16, 16), jnp.int32),
            pltpu.SemaphoreType.DMA,
        ],
        compiler_params=cp,
    )


def _combine_body(bins_ref, score_ref):
    j = jnp.sum(bins_ref[...], axis=(0, 2)).astype(jnp.float32)
    inter = jnp.stack([j[:, 0], j[:, 5], j[:, 10], j[:, 15]], axis=1)
    p = [j[:, 4 * c] + j[:, 4 * c + 1] + j[:, 4 * c + 2] + j[:, 4 * c + 3]
         for c in range(4)]
    t = [j[:, c] + j[:, 4 + c] + j[:, 8 + c] + j[:, 12 + c]
         for c in range(4)]
    card = (jnp.stack(p, axis=1) + jnp.stack(t, axis=1))
    score_ref[...] = jnp.mean(
        2.0 * inter / jnp.maximum(card, 1.0), axis=0, keepdims=True)


def kernel(output, target):
    o2 = output.reshape(8, 4, 512 * 512)
    t2 = target.reshape(8, 512 * 512)
    bins = _make_sc_kernel()(o2, t2)
    score = pl.pallas_call(
        _combine_body,
        grid=(1,),
        in_specs=[pl.BlockSpec((_NW, 8, 16, 16), lambda i: (0, 0, 0, 0))],
        out_specs=pl.BlockSpec((1, 4), lambda i: (0, 0)),
        out_shape=jax.ShapeDtypeStruct((1, 4), jnp.float32),
    )(bins)
    return score[0]


# speedup vs baseline: 8.8534x; 8.8534x over previous
import jax
import jax.numpy as jnp
from jax.experimental import pallas as pl

_NPIX = float(512 * 512)
_RC = 8
_NCHUNK = 512 // _RC

_G = [0,
      (1 << 8) + (1 << 24),
      1 + (2 << 24),
      1 + (1 << 8) + (1 << 16) + (3 << 24)]
_H = [1,
      1 + (1 << 16),
      1 + (1 << 8),
      1 + (1 << 8) + (1 << 16) + (1 << 24)]


def _dice_body(o_ref, t_ref, acc_ref, score_ref):
    b = pl.program_id(0)
    nb = pl.num_programs(0)
    i32 = jnp.int32

    def half(r, u, carry):
        ag, am, ata, atb, atab = carry
        rs = pl.ds(r * (4 * _RC) + u * _RC, _RC)
        o0 = o_ref[0, 0, rs, :]
        o1 = o_ref[0, 1, rs, :]
        o2 = o_ref[0, 2, rs, :]
        o3 = o_ref[0, 3, rs, :]
        t = t_ref[0, 0, rs, :]

        gt1 = o1 > o0
        gt3 = o3 > o2
        gtb = jnp.maximum(o2, o3) > jnp.maximum(o0, o1)
        g = jnp.where(gtb,
                      jnp.where(gt3, i32(_G[3]), i32(_G[2])),
                      jnp.where(gt1, i32(_G[1]), i32(_G[0])))
        h = jnp.where(gtb,
                      jnp.where(gt3, i32(_H[3]), i32(_H[2])),
                      jnp.where(gt1, i32(_H[1]), i32(_H[0])))
        mp = jnp.where((g >> 24) == t, h, i32(0))
        ta = t >> 1
        tb = t & 1
        return (ag + g, am + mp, ata + ta, atb + tb, atab + (ta & tb))

    def chunk(r, carry):
        for u in range(4):
            carry = half(r, u, carry)
        return carry

    zeros = jnp.zeros((_RC, 512), jnp.int32)
    ag, am, ata, atb, atab = jax.lax.fori_loop(
        0, _NCHUNK // 4, chunk, (zeros, zeros, zeros, zeros, zeros))

    m8 = i32(0xFF)
    acc_ref[0, b] = ag & m8
    acc_ref[1, b] = (ag >> 8) & m8
    acc_ref[2, b] = (ag >> 16) & m8
    acc_ref[3, b] = ata
    acc_ref[4, b] = atb
    acc_ref[5, b] = atab
    acc_ref[6, b] = am & m8
    acc_ref[7, b] = (am >> 8) & m8
    acc_ref[8, b] = (am >> 16) & m8
    acc_ref[9, b] = (am >> 24) & m8

    @pl.when(b == nb - 1)
    def _():
        st = jnp.sum(acc_ref[...], axis=(2, 3)).astype(jnp.float32)
        sa, sb, sab = st[0], st[1], st[2]
        sta, stb, stab = st[3], st[4], st[5]
        sm, sma, smb, smab = st[6], st[7], st[8], st[9]
        p3, p2, p1 = sab, sa - sab, sb - sab
        p0 = _NPIX - sa - sb + sab
        t3, t2, t1 = stab, sta - stab, stb - stab
        t0 = _NPIX - sta - stb + stab
        i3, i2, i1 = smab, sma - smab, smb - smab
        i0 = sm - sma - smb + smab
        inter = jnp.stack([i0, i1, i2, i3], axis=1)
        card = (jnp.stack([p0, p1, p2, p3], axis=1)
                + jnp.stack([t0, t1, t2, t3], axis=1))
        score_ref[...] = jnp.mean(
            2.0 * inter / jnp.maximum(card, 1.0), axis=0, keepdims=True)


def kernel(output, target):
    _, score = pl.pallas_call(
        _dice_body,
        grid=(8,),
        in_specs=[
            pl.BlockSpec((1, 4, 512, 512), lambda i: (i, 0, 0, 0)),
            pl.BlockSpec((1, 1, 512, 512), lambda i: (i, 0, 0, 0)),
        ],
        out_specs=[
            pl.BlockSpec((10, 8, _RC, 512), lambda i: (0, 0, 0, 0)),
            pl.BlockSpec((1, 4), lambda i: (0, 0)),
        ],
        out_shape=[
            jax.ShapeDtypeStruct((10, 8, _RC, 512), jnp.int32),
            jax.ShapeDtypeStruct((1, 4), jnp.float32),
        ],
    )(output, target)
    return score[0]


# speedup vs baseline: 8.9332x; 1.0090x over previous
import jax
import jax.numpy as jnp
from jax.experimental import pallas as pl

_NPIX = float(512 * 512)
_RC = 8
_NCHUNK = 512 // _RC

_G = [0,
      (1 << 8) + (1 << 24),
      1 + (2 << 24),
      1 + (1 << 8) + (1 << 16) + (3 << 24)]
_H = [1,
      1 + (1 << 16),
      1 + (1 << 8),
      1 + (1 << 8) + (1 << 16) + (1 << 24)]


def _dice_body(o_ref, t_ref, acc_ref, score_ref):
    b = pl.program_id(0)
    nb = pl.num_programs(0)
    i32 = jnp.int32

    def half(r, u, carry):
        ag, am, ata, atb, atab = carry
        rs = pl.ds(r * (8 * _RC) + u * _RC, _RC)
        o0 = o_ref[0, 0, rs, :]
        o1 = o_ref[0, 1, rs, :]
        o2 = o_ref[0, 2, rs, :]
        o3 = o_ref[0, 3, rs, :]
        t = t_ref[0, 0, rs, :]

        gt1 = o1 > o0
        gt3 = o3 > o2
        gtb = jnp.maximum(o2, o3) > jnp.maximum(o0, o1)
        g = jnp.where(gtb,
                      jnp.where(gt3, i32(_G[3]), i32(_G[2])),
                      jnp.where(gt1, i32(_G[1]), i32(_G[0])))
        h = jnp.where(gtb,
                      jnp.where(gt3, i32(_H[3]), i32(_H[2])),
                      jnp.where(gt1, i32(_H[1]), i32(_H[0])))
        mp = jnp.where((g >> 24) == t, h, i32(0))
        ta = t >> 1
        tb = t & 1
        return (ag + g, am + mp, ata + ta, atb + tb, atab + (ta & tb))

    def chunk(r, carry):
        for u in range(8):
            carry = half(r, u, carry)
        return carry

    zeros = jnp.zeros((_RC, 512), jnp.int32)
    ag, am, ata, atb, atab = jax.lax.fori_loop(
        0, _NCHUNK // 8, chunk, (zeros, zeros, zeros, zeros, zeros))

    m8 = i32(0xFF)
    acc_ref[0, b] = ag & m8
    acc_ref[1, b] = (ag >> 8) & m8
    acc_ref[2, b] = (ag >> 16) & m8
    acc_ref[3, b] = ata
    acc_ref[4, b] = atb
    acc_ref[5, b] = atab
    acc_ref[6, b] = am & m8
    acc_ref[7, b] = (am >> 8) & m8
    acc_ref[8, b] = (am >> 16) & m8
    acc_ref[9, b] = (am >> 24) & m8

    @pl.when(b == nb - 1)
    def _():
        st = jnp.sum(acc_ref[...], axis=(2, 3)).astype(jnp.float32)
        sa, sb, sab = st[0], st[1], st[2]
        sta, stb, stab = st[3], st[4], st[5]
        sm, sma, smb, smab = st[6], st[7], st[8], st[9]
        p3, p2, p1 = sab, sa - sab, sb - sab
        p0 = _NPIX - sa - sb + sab
        t3, t2, t1 = stab, sta - stab, stb - stab
        t0 = _NPIX - sta - stb + stab
        i3, i2, i1 = smab, sma - smab, smb - smab
        i0 = sm - sma - smb + smab
        inter = jnp.stack([i0, i1, i2, i3], axis=1)
        card = (jnp.stack([p0, p1, p2, p3], axis=1)
                + jnp.stack([t0, t1, t2, t3], axis=1))
        score_ref[...] = jnp.mean(
            2.0 * inter / jnp.maximum(card, 1.0), axis=0, keepdims=True)


def kernel(output, target):
    _, score = pl.pallas_call(
        _dice_body,
        grid=(8,),
        in_specs=[
            pl.BlockSpec((1, 4, 512, 512), lambda i: (i, 0, 0, 0)),
            pl.BlockSpec((1, 1, 512, 512), lambda i: (i, 0, 0, 0)),
        ],
        out_specs=[
            pl.BlockSpec((10, 8, _RC, 512), lambda i: (0, 0, 0, 0)),
            pl.BlockSpec((1, 4), lambda i: (0, 0)),
        ],
        out_shape=[
            jax.ShapeDtypeStruct((10, 8, _RC, 512), jnp.int32),
            jax.ShapeDtypeStruct((1, 4), jnp.float32),
        ],
    )(output, target)
    return score[0]
